# baseline (speedup 1.0000x reference)
import functools

import jax
import jax.numpy as jnp
from jax import lax
from jax.experimental import pallas as pl
from jax.experimental.pallas import tpu as pltpu
from jax.experimental.pallas import tpu_sc as plsc

BATCH = 4096
SEQ = 200
EMB = 64
LANES = 16
NCOL = EMB // LANES

NC = 2
NS = 16
NW = NC * NS
B_PER_W = BATCH // NW
CHUNK = 128
PAD_SEQ = 2 * CHUNK
CHUNKS_PER_B = PAD_SEQ // CHUNK
ROWS_PER_W = B_PER_W * CHUNKS_PER_B

_NEG = float(jnp.finfo(jnp.float32).min)


def _reduce_chunk(buf, off_v, r):
    iota = lax.iota(jnp.int32, LANES)
    rv = jnp.full((LANES,), r, jnp.int32)

    def body(s, accs):
        sv = jnp.full((LANES,), s, jnp.int32)
        colbase = plsc.load_gather(off_v, [rv, sv]) + iota
        return tuple(
            jnp.maximum(a, plsc.load_gather(buf, [sv, colbase + (LANES * j)]))
            for j, a in enumerate(accs)
        )

    init = tuple(jnp.full((LANES,), _NEG, jnp.float32) for _ in range(NCOL))
    return lax.fori_loop(0, CHUNK, body, init)


@functools.partial(
    pl.kernel,
    out_type=jax.ShapeDtypeStruct((BATCH, EMB), jnp.float32),
    mesh=plsc.VectorSubcoreMesh(core_axis_name="c", subcore_axis_name="s"),
    compiler_params=pltpu.CompilerParams(needs_layout_passes=False),
    scratch_types=[
        pltpu.VMEM((ROWS_PER_W, CHUNK), jnp.int32),
        pltpu.VMEM((ROWS_PER_W, CHUNK), jnp.int32),
        pltpu.VMEM((CHUNK, 2 * EMB), jnp.float32),
        pltpu.VMEM((CHUNK, 2 * EMB), jnp.float32),
        pltpu.VMEM((B_PER_W, EMB), jnp.float32),
        pltpu.SemaphoreType.DMA,
        pltpu.SemaphoreType.DMA,
    ],
)
def _bow_encode(pair_hbm, off_hbm, table_hbm, out_hbm,
                idx_v, off_v, buf0, buf1, out_v, sem0, sem1):
    wid = lax.axis_index("s") * NC + lax.axis_index("c")
    base = wid * ROWS_PER_W

    pltpu.sync_copy(pair_hbm.at[pl.ds(base, ROWS_PER_W), :], idx_v)
    pltpu.sync_copy(off_hbm.at[pl.ds(base, ROWS_PER_W), :], off_v)

    pltpu.async_copy(table_hbm.at[idx_v.at[0]], buf0, sem0)
    pltpu.async_copy(table_hbm.at[idx_v.at[1]], buf1, sem1)

    def gbody(g, carry):
        r0 = 2 * g

        pltpu.make_async_copy(table_hbm.at[idx_v.at[r0]], buf0, sem0).wait()
        acc0 = _reduce_chunk(buf0, off_v, r0)

        @pl.when(g < B_PER_W - 1)
        def _():
            pltpu.async_copy(table_hbm.at[idx_v.at[r0 + 2]], buf0, sem0)

        pltpu.make_async_copy(table_hbm.at[idx_v.at[r0 + 1]], buf1, sem1).wait()
        acc1 = _reduce_chunk(buf1, off_v, r0 + 1)

        @pl.when(g < B_PER_W - 1)
        def _():
            pltpu.async_copy(table_hbm.at[idx_v.at[r0 + 3]], buf1, sem1)

        for j in range(NCOL):
            out_v[g, pl.ds(LANES * j, LANES)] = jnp.maximum(acc0[j], acc1[j])
        return carry

    lax.fori_loop(0, B_PER_W, gbody, 0)

    pltpu.sync_copy(out_v, out_hbm.at[pl.ds(wid * B_PER_W, B_PER_W), :])


@jax.jit
def kernel(input, emb_weight):
    idx = input.astype(jnp.int32)
    idx = jnp.concatenate([idx, idx[:, : PAD_SEQ - SEQ]], axis=1)
    pair = (idx >> 1).reshape(BATCH * CHUNKS_PER_B, CHUNK)
    off = ((idx & 1) << 6).reshape(BATCH * CHUNKS_PER_B, CHUNK)
    table = emb_weight.reshape(emb_weight.shape[0] // 2, 2 * EMB)
    return _bow_encode(pair, off, table)
